# Initial kernel scaffold; baseline (speedup 1.0000x reference)
import jax
import jax.numpy as jnp
from jax.experimental import pallas as pl


def kernel(x, conv_w, conv_b, cls_w, cls_b, bbox_w, bbox_b):
    raise NotImplementedError("write your pallas kernel here")


# speedup vs baseline: 1.1931x; 1.1931x over previous
import functools

import jax
import jax.numpy as jnp
from jax.experimental import pallas as pl
from jax.experimental.pallas import tpu as pltpu

L, C, H, W = 5, 256, 100, 152
A = 3
HP, WP = H + 2, W + 2
S_OUT = (H - 1) * WP + WP
CS = 1408
NC = -(-S_OUT // CS)
S_PAD = NC * CS + 512
NHEAD = 16


def _rpn_kernel(x_ref, w9_ref, cb_ref, wh_ref, hb_ref, out_ref):
    cb = cb_ref[...]
    hb = hb_ref[...]
    wh = wh_ref[...]
    for c in range(NC):
        base = c * CS
        acc = jnp.zeros((C, CS), jnp.float32)
        for dh in range(3):
            for dw in range(3):
                off = dh * WP + dw
                xs = x_ref[0, :, base + off:base + off + CS]
                acc += jnp.dot(w9_ref[dh * 3 + dw], xs,
                               preferred_element_type=jnp.float32)
        t = jnp.maximum(acc + cb, 0.0).astype(jnp.bfloat16)
        out_ref[0, :, base:base + CS] = (
            jnp.dot(wh, t, preferred_element_type=jnp.float32) + hb)


@jax.jit
def kernel(x, conv_w, conv_b, cls_w, cls_b, bbox_w, bbox_b):
    xp = jnp.pad(x, ((0, 0), (0, 0), (1, 1), (1, 1)))
    xp = xp.reshape(L, C, HP * WP)
    xp = jnp.pad(xp, ((0, 0), (0, 0), (0, S_PAD - HP * WP)))
    xp = xp.astype(jnp.bfloat16)

    w9 = jnp.transpose(conv_w, (2, 3, 0, 1)).reshape(9, C, C)
    w9 = w9.astype(jnp.bfloat16)
    wh = jnp.concatenate([cls_w[:, :, 0, 0], bbox_w[:, :, 0, 0],
                          jnp.zeros((NHEAD - A - 4 * A, C))], axis=0)
    wh = wh.astype(jnp.bfloat16)
    hb = jnp.concatenate([cls_b, bbox_b,
                          jnp.zeros((NHEAD - A - 4 * A,))])[:, None]
    cb = conv_b[:, None]

    out = pl.pallas_call(
        _rpn_kernel,
        grid=(L,),
        in_specs=[
            pl.BlockSpec((1, C, S_PAD), lambda l: (l, 0, 0)),
            pl.BlockSpec((9, C, C), lambda l: (0, 0, 0)),
            pl.BlockSpec((C, 1), lambda l: (0, 0)),
            pl.BlockSpec((NHEAD, C), lambda l: (0, 0)),
            pl.BlockSpec((NHEAD, 1), lambda l: (0, 0)),
        ],
        out_specs=pl.BlockSpec((1, NHEAD, NC * CS), lambda l: (l, 0, 0)),
        out_shape=jax.ShapeDtypeStruct((L, NHEAD, NC * CS), jnp.float32),
    )(xp, w9, cb, wh, hb)

    r = out[:, :, :H * WP].reshape(L, NHEAD, H, WP)[:, :, :, :W]
    return (r[:, :A], r[:, A:A + 4 * A])


# speedup vs baseline: 1.7621x; 1.4770x over previous
import jax
import jax.numpy as jnp
from jax.experimental import pallas as pl
from jax.experimental.pallas import tpu as pltpu

L, C, H, W = 5, 256, 100, 152
A = 3
HP, WP = H + 2, W + 2
S_PAD = -(-(HP * WP) // 128) * 128
RB = 10
NB = H // RB
BS = RB * WP
NHEAD = 16


def _rpn_kernel(x_ref, w9_ref, cb_ref, wh_ref, hb_ref,
                cls_ref, bbox_ref, xs_ref):
    lvl = pl.program_id(0)

    @pl.when(lvl == 0)
    def _zero():
        xs_ref[...] = jnp.zeros((C, S_PAD), jnp.bfloat16)

    for h in range(H):
        xs_ref[:, (h + 1) * WP + 1:(h + 1) * WP + 1 + W] = (
            x_ref[0, :, h * W:(h + 1) * W].astype(jnp.bfloat16))

    cb = cb_ref[...]
    hb = hb_ref[...]
    wh = wh_ref[...]
    for b in range(NB):
        h0 = b * RB
        acc = jnp.zeros((C, BS), jnp.float32)
        for dh in range(3):
            for dw in range(3):
                start = (h0 + dh) * WP + dw
                acc += jnp.dot(w9_ref[dh * 3 + dw],
                               xs_ref[:, start:start + BS],
                               preferred_element_type=jnp.float32)
        t = jnp.maximum(acc + cb, 0.0).astype(jnp.bfloat16)
        o = jnp.dot(wh, t, preferred_element_type=jnp.float32) + hb
        for r in range(RB):
            row = o[:, r * WP:r * WP + W]
            cls_ref[0, :, h0 + r, :] = row[:A]
            bbox_ref[0, :, h0 + r, :] = row[A:A + 4 * A]


@jax.jit
def kernel(x, conv_w, conv_b, cls_w, cls_b, bbox_w, bbox_b):
    xf = x.reshape(L, C, H * W)
    w9 = jnp.transpose(conv_w, (2, 3, 0, 1)).reshape(9, C, C)
    w9 = w9.astype(jnp.bfloat16)
    wh = jnp.concatenate([cls_w[:, :, 0, 0], bbox_w[:, :, 0, 0],
                          jnp.zeros((NHEAD - A - 4 * A, C))], axis=0)
    wh = wh.astype(jnp.bfloat16)
    hb = jnp.concatenate([cls_b, bbox_b,
                          jnp.zeros((NHEAD - A - 4 * A,))])[:, None]
    cb = conv_b[:, None]

    scores, bbox = pl.pallas_call(
        _rpn_kernel,
        grid=(L,),
        in_specs=[
            pl.BlockSpec((1, C, H * W), lambda l: (l, 0, 0)),
            pl.BlockSpec((9, C, C), lambda l: (0, 0, 0)),
            pl.BlockSpec((C, 1), lambda l: (0, 0)),
            pl.BlockSpec((NHEAD, C), lambda l: (0, 0)),
            pl.BlockSpec((NHEAD, 1), lambda l: (0, 0)),
        ],
        out_specs=[
            pl.BlockSpec((1, A, H, W), lambda l: (l, 0, 0, 0)),
            pl.BlockSpec((1, 4 * A, H, W), lambda l: (l, 0, 0, 0)),
        ],
        out_shape=[
            jax.ShapeDtypeStruct((L, A, H, W), jnp.float32),
            jax.ShapeDtypeStruct((L, 4 * A, H, W), jnp.float32),
        ],
        scratch_shapes=[pltpu.VMEM((C, S_PAD), jnp.bfloat16)],
    )(xf, w9, cb, wh, hb)

    return (scores, bbox)
